# baseline (speedup 1.0000x reference)
import functools

import jax
import jax.numpy as jnp
from jax import lax
from jax.experimental import pallas as pl
from jax.experimental.pallas import tpu as pltpu
from jax.experimental.pallas import tpu_sc as plsc

_PSIZES = (41, 37, 31, 23)
_D = 16
_NT = 4
_B = 16384
_NC = 2
_NS = 16
_NW = _NC * _NS
_BPW = _B // _NW
_L = 16
_PR = 48
_PCOL = 128

_GATHER_DNUMS = lax.GatherDimensionNumbers(
    offset_dims=(), collapsed_slice_dims=(0,), start_index_map=(0,)
)


def _bcast_lane(vec, j):
    jj = jnp.full((_L, 1), j, jnp.int32)
    return lax.gather(
        vec, jj, _GATHER_DNUMS, (1,),
        mode=lax.GatherScatterMode.PROMISE_IN_BOUNDS,
    )


def _body(uid_hbm, w_hbm, out_hbm, uid_v, w_v, out_v, sem):
    wid = lax.axis_index("s") * _NC + lax.axis_index("c")
    base = wid * _BPW

    with jax.named_scope("stage"):
        cp_uid = pltpu.async_copy(uid_hbm.at[pl.ds(base, _BPW)], uid_v, sem)
        cp_w = pltpu.async_copy(w_hbm, w_v, sem)
        cp_uid.wait()
        cp_w.wait()

    lanes = lax.iota(jnp.int32, _L)
    lane_off = [lanes + t * _D for t in range(_NT)]

    def g_step(i):
        u = uid_v[pl.ds(i, _L)]
        uf = u.astype(jnp.float32)
        idxs = []
        for t, p in enumerate(_PSIZES):
            q = (uf * (1.0 / p)).astype(jnp.int32).astype(jnp.float32)
            r = uf - q * float(p)
            r = jnp.where(r < 0.0, r + p, r)
            r = jnp.where(r >= p, r - p, r)
            idxs.append(r.astype(jnp.int32))
        for j in range(_L):
            for t in range(_NT):
                row = _bcast_lane(idxs[t], j)
                vals = plsc.load_gather(w_v, [row, lane_off[t]])
                out_v[i + j, pl.ds(t * _D, _D)] = vals

    with jax.named_scope("lookup"):
        plsc.parallel_loop(0, _BPW, step=_L, unroll=2)(g_step)

    with jax.named_scope("writeback"):
        pltpu.sync_copy(out_v, out_hbm.at[pl.ds(base, _BPW)])


@functools.partial(
    pl.kernel,
    out_type=jax.ShapeDtypeStruct((_B, _NT * _D), jnp.float32),
    mesh=plsc.VectorSubcoreMesh(core_axis_name="c", subcore_axis_name="s"),
    scratch_types=[
        pltpu.VMEM((_BPW,), jnp.int32),
        pltpu.VMEM((_PR, _PCOL), jnp.float32),
        pltpu.VMEM((_BPW, _NT * _D), jnp.float32),
        pltpu.SemaphoreType.DMA,
    ],
    compiler_params=pltpu.CompilerParams(
        use_tc_tiling_on_sc=True,
        needs_layout_passes=False,
        disable_bounds_checks=True,
    ),
)
def _sc_lookup(uid_hbm, w_hbm, out_hbm, uid_v, w_v, out_v, sem):
    _body(uid_hbm, w_hbm, out_hbm, uid_v, w_v, out_v, sem)


def kernel(user_ids, W0, W1, W2, W3):
    w_pack = jnp.zeros((_PR, _PCOL), jnp.float32)
    for t, w in enumerate((W0, W1, W2, W3)):
        w_pack = w_pack.at[: w.shape[0], t * _D : (t + 1) * _D].set(w)
    return _sc_lookup(user_ids.astype(jnp.int32), w_pack)


# speedup vs baseline: 1.5034x; 1.5034x over previous
import functools

import jax
import jax.numpy as jnp
from jax import lax
from jax.experimental import pallas as pl
from jax.experimental.pallas import tpu as pltpu
from jax.experimental.pallas import tpu_sc as plsc

_PSIZES = (41, 37, 31, 23)
_D = 16
_NT = 4
_B = 16384
_NC = 2
_NS = 16
_NW = _NC * _NS
_BPW = _B // _NW
_L = 16
_PR = 48
_PCOL = 128

_GATHER_DNUMS = lax.GatherDimensionNumbers(
    offset_dims=(), collapsed_slice_dims=(0,), start_index_map=(0,)
)


def _bcast_lane(vec, j):
    jj = jnp.full((_L, 1), j, jnp.int32)
    return lax.gather(
        vec, jj, _GATHER_DNUMS, (1,),
        mode=lax.GatherScatterMode.PROMISE_IN_BOUNDS,
    )


def _body(uid_hbm, w_hbm, out_hbm, uid_v, w_v, out_v, sem):
    wid = lax.axis_index("s") * _NC + lax.axis_index("c")
    base = wid * _BPW

    with jax.named_scope("stage"):
        cp_uid = pltpu.async_copy(uid_hbm.at[pl.ds(base, _BPW)], uid_v, sem)
        cp_w = pltpu.async_copy(w_hbm, w_v, sem)
        cp_uid.wait()
        cp_w.wait()

    lanes = lax.iota(jnp.int32, _L)
    lane_off = [lanes + t * _D for t in range(_NT)]

    def g_step(i):
        u = uid_v[pl.ds(i, _L)]
        uf = u.astype(jnp.float32)
        idxs = []
        for t, p in enumerate(_PSIZES):
            q = (uf * (1.0 / p)).astype(jnp.int32).astype(jnp.float32)
            r = uf - q * float(p)
            r = jnp.where(r < 0.0, r + p, r)
            r = jnp.where(r >= p, r - p, r)
            idxs.append(r.astype(jnp.int32))
        for j in range(_L):
            for t in range(_NT):
                row = _bcast_lane(idxs[t], j)
                vals = plsc.load_gather(w_v, [row, lane_off[t]])
                out_v[i + j, pl.ds(t * _D, _D)] = vals

    with jax.named_scope("lookup"):
        plsc.parallel_loop(0, _BPW, step=_L, unroll=2)(g_step)

    with jax.named_scope("writeback"):
        pltpu.sync_copy(out_v, out_hbm.at[pl.ds(base, _BPW)])


@functools.partial(
    pl.kernel,
    out_type=jax.ShapeDtypeStruct((_B, _NT * _D), jnp.float32),
    mesh=plsc.VectorSubcoreMesh(core_axis_name="c", subcore_axis_name="s"),
    scratch_types=[
        pltpu.VMEM((_BPW,), jnp.int32),
        pltpu.VMEM((_PR, _PCOL), jnp.float32),
        pltpu.VMEM((_BPW, _NT * _D), jnp.float32),
        pltpu.SemaphoreType.DMA,
    ],
    compiler_params=pltpu.CompilerParams(
        use_tc_tiling_on_sc=True,
        needs_layout_passes=False,
        disable_bounds_checks=True,
    ),
)
def _sc_lookup(uid_hbm, w_hbm, out_hbm, uid_v, w_v, out_v, sem):
    _body(uid_hbm, w_hbm, out_hbm, uid_v, w_v, out_v, sem)


def kernel(user_ids, W0, W1, W2, W3):
    cols = jnp.concatenate(
        [
            jnp.pad(w, ((0, _PR - w.shape[0]), (0, 0)))
            for w in (W0, W1, W2, W3)
        ],
        axis=1,
    )
    w_pack = jnp.pad(cols, ((0, 0), (0, _PCOL - _NT * _D)))
    return _sc_lookup(user_ids.astype(jnp.int32), w_pack)


# speedup vs baseline: 1.5483x; 1.0299x over previous
import functools

import jax
import jax.numpy as jnp
from jax import lax
from jax.experimental import pallas as pl
from jax.experimental.pallas import tpu as pltpu
from jax.experimental.pallas import tpu_sc as plsc

_PSIZES = (41, 37, 31, 23)
_D = 16
_NT = 4
_B = 16384
_NC = 2
_NS = 16
_NW = _NC * _NS
_BPW = _B // _NW
_L = 16
_PR = 48
_PCOL = 128

_GATHER_DNUMS = lax.GatherDimensionNumbers(
    offset_dims=(), collapsed_slice_dims=(0,), start_index_map=(0,)
)


def _bcast_lane(vec, j):
    jj = jnp.full((_L, 1), j, jnp.int32)
    return lax.gather(
        vec, jj, _GATHER_DNUMS, (1,),
        mode=lax.GatherScatterMode.PROMISE_IN_BOUNDS,
    )


def _body(uid_hbm, w_hbm, out_hbm, uid_v, w_v, out_v, sem):
    wid = lax.axis_index("s") * _NC + lax.axis_index("c")
    base = wid * _BPW

    with jax.named_scope("stage"):
        cp_uid = pltpu.async_copy(uid_hbm.at[pl.ds(base, _BPW)], uid_v, sem)
        cp_w = pltpu.async_copy(w_hbm, w_v, sem)
        cp_uid.wait()
        cp_w.wait()

    lanes = lax.iota(jnp.int32, _L)
    lane_off = [lanes + t * _D for t in range(_NT)]

    def g_step(i):
        u = uid_v[pl.ds(i, _L)]
        uf = u.astype(jnp.float32)
        idxs = []
        for t, p in enumerate(_PSIZES):
            q = (uf * (1.0 / p)).astype(jnp.int32).astype(jnp.float32)
            r = uf - q * float(p)
            r = jnp.where(r < 0.0, r + p, r)
            r = jnp.where(r >= p, r - p, r)
            idxs.append(r.astype(jnp.int32))
        for j in range(_L):
            for t in range(_NT):
                row = _bcast_lane(idxs[t], j)
                vals = plsc.load_gather(w_v, [row, lane_off[t]])
                out_v[i + j, pl.ds(t * _D, _D)] = vals

    with jax.named_scope("lookup"):
        plsc.parallel_loop(0, _BPW, step=_L, unroll=4)(g_step)

    with jax.named_scope("writeback"):
        pltpu.sync_copy(out_v, out_hbm.at[pl.ds(base, _BPW)])


@functools.partial(
    pl.kernel,
    out_type=jax.ShapeDtypeStruct((_B, _NT * _D), jnp.float32),
    mesh=plsc.VectorSubcoreMesh(core_axis_name="c", subcore_axis_name="s"),
    scratch_types=[
        pltpu.VMEM((_BPW,), jnp.int32),
        pltpu.VMEM((_PR, _PCOL), jnp.float32),
        pltpu.VMEM((_BPW, _NT * _D), jnp.float32),
        pltpu.SemaphoreType.DMA,
    ],
    compiler_params=pltpu.CompilerParams(
        use_tc_tiling_on_sc=True,
        needs_layout_passes=False,
        disable_bounds_checks=True,
    ),
)
def _sc_lookup(uid_hbm, w_hbm, out_hbm, uid_v, w_v, out_v, sem):
    _body(uid_hbm, w_hbm, out_hbm, uid_v, w_v, out_v, sem)


def kernel(user_ids, W0, W1, W2, W3):
    cols = jnp.concatenate(
        [
            jnp.pad(w, ((0, _PR - w.shape[0]), (0, 0)))
            for w in (W0, W1, W2, W3)
        ],
        axis=1,
    )
    w_pack = jnp.pad(cols, ((0, 0), (0, _PCOL - _NT * _D)))
    return _sc_lookup(user_ids.astype(jnp.int32), w_pack)


# speedup vs baseline: 1.5793x; 1.0200x over previous
import functools

import jax
import jax.numpy as jnp
from jax import lax
from jax.experimental import pallas as pl
from jax.experimental.pallas import tpu as pltpu
from jax.experimental.pallas import tpu_sc as plsc

_PSIZES = (41, 37, 31, 23)
_D = 16
_NT = 4
_B = 16384
_NC = 2
_NS = 16
_NW = _NC * _NS
_BPW = _B // _NW
_L = 16
_PR = 48
_PCOL = 128

_GATHER_DNUMS = lax.GatherDimensionNumbers(
    offset_dims=(), collapsed_slice_dims=(0,), start_index_map=(0,)
)


def _bcast_lane(vec, j):
    jj = jnp.full((_L, 1), j, jnp.int32)
    return lax.gather(
        vec, jj, _GATHER_DNUMS, (1,),
        mode=lax.GatherScatterMode.PROMISE_IN_BOUNDS,
    )


def _body(uid_hbm, w_hbm, out_hbm, uid_v, w_v, out_v, sem):
    wid = lax.axis_index("s") * _NC + lax.axis_index("c")
    base = wid * _BPW

    with jax.named_scope("stage"):
        cp_uid = pltpu.async_copy(uid_hbm.at[pl.ds(base, _BPW)], uid_v, sem)
        cp_w = pltpu.async_copy(w_hbm, w_v, sem)
        cp_uid.wait()
        cp_w.wait()

    lanes = lax.iota(jnp.int32, _L)
    lane_off = [lanes + t * _D for t in range(_NT)]

    def g_step(i):
        u = uid_v[pl.ds(i, _L)]
        uf = u.astype(jnp.float32)
        idxs = []
        for t, p in enumerate(_PSIZES):
            q = (uf * (1.0 / p)).astype(jnp.int32).astype(jnp.float32)
            r = uf - q * float(p)
            r = jnp.where(r < 0.0, r + p, r)
            r = jnp.where(r >= p, r - p, r)
            idxs.append(r.astype(jnp.int32))
        for j in range(_L):
            for t in range(_NT):
                row = _bcast_lane(idxs[t], j)
                vals = plsc.load_gather(w_v, [row, lane_off[t]])
                out_v[i + j, pl.ds(t * _D, _D)] = vals

    half = _BPW // 2
    with jax.named_scope("lookup"):
        plsc.parallel_loop(0, half, step=_L, unroll=4)(g_step)
        cp0 = pltpu.async_copy(
            out_v.at[pl.ds(0, half)], out_hbm.at[pl.ds(base, half)], sem
        )
        plsc.parallel_loop(half, _BPW, step=_L, unroll=4)(g_step)
        cp1 = pltpu.async_copy(
            out_v.at[pl.ds(half, half)],
            out_hbm.at[pl.ds(base + half, half)],
            sem,
        )

    with jax.named_scope("writeback"):
        cp0.wait()
        cp1.wait()


@functools.partial(
    pl.kernel,
    out_type=jax.ShapeDtypeStruct((_B, _NT * _D), jnp.float32),
    mesh=plsc.VectorSubcoreMesh(core_axis_name="c", subcore_axis_name="s"),
    scratch_types=[
        pltpu.VMEM((_BPW,), jnp.int32),
        pltpu.VMEM((_PR, _PCOL), jnp.float32),
        pltpu.VMEM((_BPW, _NT * _D), jnp.float32),
        pltpu.SemaphoreType.DMA,
    ],
    compiler_params=pltpu.CompilerParams(
        use_tc_tiling_on_sc=True,
        needs_layout_passes=False,
        disable_bounds_checks=True,
    ),
)
def _sc_lookup(uid_hbm, w_hbm, out_hbm, uid_v, w_v, out_v, sem):
    _body(uid_hbm, w_hbm, out_hbm, uid_v, w_v, out_v, sem)


def kernel(user_ids, W0, W1, W2, W3):
    cols = jnp.concatenate(
        [
            jnp.pad(w, ((0, _PR - w.shape[0]), (0, 0)))
            for w in (W0, W1, W2, W3)
        ],
        axis=1,
    )
    w_pack = jnp.pad(cols, ((0, 0), (0, _PCOL - _NT * _D)))
    return _sc_lookup(user_ids.astype(jnp.int32), w_pack)
